# baseline (speedup 1.0000x reference)
import functools
import math

import jax
import jax.numpy as jnp
from jax import lax
from jax.experimental import pallas as pl

N = 1024
IN_DIM = 128
HID = 128


def _layer(x, neg_mask, Wq, bq, Wk, bk, Wv, bv, Ws, bs):
    q = jnp.dot(x, Wq, preferred_element_type=jnp.float32) + bq
    k = jnp.dot(x, Wk, preferred_element_type=jnp.float32) + bk
    v = jnp.dot(x, Wv, preferred_element_type=jnp.float32) + bv
    s = jnp.dot(x, Ws, preferred_element_type=jnp.float32) + bs
    logits = lax.dot_general(k, q, (((1,), (1,)), ((), ())),
                             preferred_element_type=jnp.float32)
    masked = logits + neg_mask
    amax = jnp.max(masked, axis=0)
    amax = jnp.where(jnp.isfinite(amax), amax, 0.0)
    ex = jnp.exp(masked - amax[None, :])
    denom = jnp.sum(ex, axis=0)
    agg = lax.dot_general(ex, v, (((0,), (0,)), ((), ())),
                          preferred_element_type=jnp.float32)
    out = agg * (1.0 / (denom[:, None] + 1e-16))
    return out + s


def _block_kernel(x_ref, adj_ref,
                  wq1, bq1, wk1, bk1, wv1, bv1, ws1, bs1,
                  wq2, bq2, wk2, bk2, wv2, bv2, ws2, bs2,
                  out_ref):
    x = x_ref[:]
    neg_mask = jnp.where(adj_ref[:] != 0.0, 0.0, -jnp.inf)
    h1 = _layer(x, neg_mask,
                wq1[:], bq1[:], wk1[:], bk1[:], wv1[:], bv1[:], ws1[:], bs1[:])
    h1 = jnp.where(h1 > 0.0, h1, jnp.exp(jnp.minimum(h1, 0.0)) - 1.0)
    out_ref[:] = _layer(h1, neg_mask,
                        wq2[:], bq2[:], wk2[:], bk2[:], wv2[:], bv2[:],
                        ws2[:], bs2[:])


@jax.jit
def kernel(x, XY_Adj, Wq1, bq1, Wk1, bk1, Wv1, bv1, Ws1, bs1,
           Wq2, bq2, Wk2, bk2, Wv2, bv2, Ws2, bs2):
    scale1 = 1.0 / math.sqrt(float(Wq1.shape[1]))
    scale2 = 1.0 / math.sqrt(float(Wq2.shape[1]))
    Wq1 = Wq1 * scale1
    bq1 = bq1 * scale1
    Wq2 = Wq2 * scale2
    bq2 = bq2 * scale2
    biases = [b.reshape(1, -1) for b in (bq1, bk1, bv1, bs1, bq2, bk2, bv2, bs2)]
    bq1, bk1, bv1, bs1, bq2, bk2, bv2, bs2 = biases
    return pl.pallas_call(
        _block_kernel,
        out_shape=jax.ShapeDtypeStruct((N, IN_DIM), jnp.float32),
    )(x, XY_Adj,
      Wq1, bq1, Wk1, bk1, Wv1, bv1, Ws1, bs1,
      Wq2, bq2, Wk2, bk2, Wv2, bv2, Ws2, bs2)


# speedup vs baseline: 1.3103x; 1.3103x over previous
import functools
import math

import jax
import jax.numpy as jnp
from jax import lax
from jax.experimental import pallas as pl

N = 1024
IN_DIM = 128
HID = 128


def _layer(x, neg_mask, Wq, bq, Wk, bk, Wv, bv, Ws, bs):
    scale = 1.0 / math.sqrt(float(Wq.shape[1]))
    Wq = Wq * scale
    bq = bq * scale
    q = jnp.dot(x, Wq, preferred_element_type=jnp.float32) + bq
    k = jnp.dot(x, Wk, preferred_element_type=jnp.float32) + bk
    v = jnp.dot(x, Wv, preferred_element_type=jnp.float32) + bv
    s = jnp.dot(x, Ws, preferred_element_type=jnp.float32) + bs
    logits = lax.dot_general(k, q, (((1,), (1,)), ((), ())),
                             preferred_element_type=jnp.float32)
    masked = logits + neg_mask
    amax = jnp.max(masked, axis=0)
    amax = jnp.where(jnp.isfinite(amax), amax, 0.0)
    ex = jnp.exp(masked - amax[None, :])
    denom = jnp.sum(ex, axis=0)
    agg = lax.dot_general(ex, v, (((0,), (0,)), ((), ())),
                          preferred_element_type=jnp.float32)
    out = agg * (1.0 / (denom[:, None] + 1e-16))
    return out + s


def _block_kernel(x_ref, adj_ref,
                  wq1, bq1, wk1, bk1, wv1, bv1, ws1, bs1,
                  wq2, bq2, wk2, bk2, wv2, bv2, ws2, bs2,
                  out_ref):
    x = x_ref[:]
    neg_mask = jnp.where(adj_ref[:] != 0.0, 0.0, -jnp.inf)
    h1 = _layer(x, neg_mask,
                wq1[:], bq1[:], wk1[:], bk1[:], wv1[:], bv1[:], ws1[:], bs1[:])
    h1 = jnp.where(h1 > 0.0, h1, jnp.exp(jnp.minimum(h1, 0.0)) - 1.0)
    out_ref[:] = _layer(h1, neg_mask,
                        wq2[:], bq2[:], wk2[:], bk2[:], wv2[:], bv2[:],
                        ws2[:], bs2[:])


@jax.jit
def kernel(x, XY_Adj, Wq1, bq1, Wk1, bk1, Wv1, bv1, Ws1, bs1,
           Wq2, bq2, Wk2, bk2, Wv2, bv2, Ws2, bs2):
    biases = [b.reshape(1, -1) for b in (bq1, bk1, bv1, bs1, bq2, bk2, bv2, bs2)]
    bq1, bk1, bv1, bs1, bq2, bk2, bv2, bs2 = biases
    return pl.pallas_call(
        _block_kernel,
        out_shape=jax.ShapeDtypeStruct((N, IN_DIM), jnp.float32),
    )(x, XY_Adj,
      Wq1, bq1, Wk1, bk1, Wv1, bv1, Ws1, bs1,
      Wq2, bq2, Wk2, bk2, Wv2, bv2, Ws2, bs2)
